# Initial kernel scaffold; baseline (speedup 1.0000x reference)
import jax
import jax.numpy as jnp
from jax.experimental import pallas as pl


def kernel(x, edge_attr, params, edge_index, batch):
    raise NotImplementedError("write your pallas kernel here")


# speedup vs baseline: 1.0971x; 1.0971x over previous
import functools

import jax
import jax.numpy as jnp
from jax import lax
from jax.experimental import pallas as pl
from jax.experimental.pallas import tpu as pltpu
from jax.experimental.pallas import tpu_sc as plsc

N = 10000
E = 320000
NG = 16
EPS = 1e-5
EB = 2560
NB = E // EB
CH = 80
EW = E // 32
NEG = -3.0e38
NP = 10240
RPT = NP // 16


@functools.cache
def _mesh():
    return plsc.VectorSubcoreMesh(core_axis_name="c", subcore_axis_name="s")


def _dot(a, b):
    return jnp.dot(a, b, preferred_element_type=jnp.float32)


def _dot_hp(a, b, dims):
    return lax.dot_general(a, b, (dims, ((), ())),
                           precision=lax.Precision.HIGHEST,
                           preferred_element_type=jnp.float32)


def _lrelu(x):
    return jnp.where(x > 0, x, 0.2 * x)


def _bn_apply(z, mu, var, gamma, beta):
    return (z - mu) / jnp.sqrt(var + EPS) * gamma + beta


def _mv(z):
    return (jnp.mean(z, axis=0, keepdims=True),
            jnp.var(z, axis=0, keepdims=True))


def _r1(a):
    return a.reshape(1, -1)


def _bnmm_body(x_ref, mu_ref, var_ref, g_ref, b_ref, w_ref, o_ref, act):
    z = _bn_apply(x_ref[...], mu_ref[...], var_ref[...], g_ref[...], b_ref[...])
    if act == 'lrelu':
        z = _lrelu(z)
    elif act == 'tanh':
        z = jnp.tanh(z)
    o_ref[...] = _dot(z, w_ref[...])


def _bnmm(x, stats, g, b, w, act='none'):
    mu, var = stats
    body = functools.partial(_bnmm_body, act=act)
    return pl.pallas_call(
        body,
        out_shape=jax.ShapeDtypeStruct((x.shape[0], w.shape[1]), jnp.float32),
    )(x, mu, var, _r1(g), _r1(b), w)


def _bnmm2_body(x_ref, mu_ref, var_ref, g_ref, b_ref, wa_ref, wb_ref,
                oa_ref, ob_ref, act):
    z = _bn_apply(x_ref[...], mu_ref[...], var_ref[...], g_ref[...], b_ref[...])
    if act == 'tanh':
        z = jnp.tanh(z)
    oa_ref[...] = _dot(z, wa_ref[...])
    ob_ref[...] = _dot(z, wb_ref[...])


def _bnmm2(x, stats, g, b, wa, wb, act='tanh'):
    mu, var = stats
    body = functools.partial(_bnmm2_body, act=act)
    return pl.pallas_call(
        body,
        out_shape=[jax.ShapeDtypeStruct((x.shape[0], wa.shape[1]), jnp.float32),
                   jax.ShapeDtypeStruct((x.shape[0], wb.shape[1]), jnp.float32)],
    )(x, mu, var, _r1(g), _r1(b), wa, wb)


def _ep_body(x_ref, p_refs, o_refs, act, nconv):
    x = x_ref[...]
    for c in range(nconv):
        mu, var, g, b, w = p_refs[5 * c:5 * c + 5]
        z = _bn_apply(x, mu[...], var[...], g[...], b[...])
        if act == 'lrelu':
            z = _lrelu(z)
        elif act == 'tanh':
            z = jnp.tanh(z)
        o_refs[c][...] = _dot(z, w[...])


def _edge_pass(x, params3, act, dout):
    din = x.shape[1]
    nconv = len(params3)

    def body(*refs):
        x_ref = refs[0]
        p_refs = refs[1:1 + 5 * nconv]
        o_refs = refs[1 + 5 * nconv:]
        _ep_body(x_ref, p_refs, o_refs, act, nconv)

    sm = lambda shape: pl.BlockSpec(shape, lambda i: (0, 0))
    in_specs = [pl.BlockSpec((EB, din), lambda i: (i, 0))]
    args = [x]
    for (mu, var, g, b, w) in params3:
        in_specs += [sm((1, din)), sm((1, din)), sm((1, din)), sm((1, din)),
                     sm(w.shape)]
        args += [mu, var, _r1(g), _r1(b), w]
    return pl.pallas_call(
        body,
        grid=(NB,),
        in_specs=in_specs,
        out_specs=[pl.BlockSpec((EB, dout), lambda i: (i, 0))] * nconv,
        out_shape=[jax.ShapeDtypeStruct((E, dout), jnp.float32)] * nconv,
    )(*args)


def _pass1_body(z2_ref, gu_ref, gv_ref, mu_ref, var_ref, g_ref, b_ref,
                w1_ref, a1_ref, d):
    eb = jnp.tanh(_bn_apply(z2_ref[...], mu_ref[...], var_ref[...],
                            g_ref[...], b_ref[...]))
    cat = jnp.concatenate([eb, gu_ref[...][:, :d], gv_ref[...][:, :d]], axis=1)
    a1_ref[...] = _dot(cat, w1_ref[...])


def _pass1(z2, gu, gv, stats, g, b, w1, d):
    mu, var = stats
    sm = lambda shape: pl.BlockSpec(shape, lambda i: (0, 0))
    return pl.pallas_call(
        functools.partial(_pass1_body, d=d),
        grid=(NB,),
        in_specs=[pl.BlockSpec((EB, 16), lambda i: (i, 0)),
                  pl.BlockSpec((EB, 128), lambda i: (i, 0)),
                  pl.BlockSpec((EB, 128), lambda i: (i, 0)),
                  sm((1, 16)), sm((1, 16)), sm((1, 16)), sm((1, 16)),
                  sm(w1.shape)],
        out_specs=pl.BlockSpec((EB, 128), lambda i: (i, 0)),
        out_shape=jax.ShapeDtypeStruct((E, 128), jnp.float32),
    )(z2, gu, gv, mu, var, _r1(g), _r1(b), w1)


def _mid_body(a_ref, mu_ref, var_ref, g_ref, b_ref, w_ref, o_ref, pad_ones):
    h = _lrelu(_bn_apply(a_ref[...], mu_ref[...], var_ref[...],
                         g_ref[...], b_ref[...]))
    o = _dot(h, w_ref[...])
    if pad_ones:
        o = jnp.concatenate(
            [o, jnp.ones((o.shape[0], 1), jnp.float32),
             jnp.zeros((o.shape[0], 127 - o.shape[1]), jnp.float32)], axis=1)
    o_ref[...] = o


def _mid_pass(a, stats, g, b, w, pad_ones=False):
    mu, var = stats
    din, dout = w.shape
    ow = 128 if pad_ones else dout
    sm = lambda shape: pl.BlockSpec(shape, lambda i: (0, 0))
    return pl.pallas_call(
        functools.partial(_mid_body, pad_ones=pad_ones),
        grid=(NB,),
        in_specs=[pl.BlockSpec((EB, din), lambda i: (i, 0)),
                  sm((1, din)), sm((1, din)), sm((1, din)), sm((1, din)),
                  sm((din, dout))],
        out_specs=pl.BlockSpec((EB, ow), lambda i: (i, 0)),
        out_shape=jax.ShapeDtypeStruct((E, ow), jnp.float32),
    )(a, stats[0], stats[1], _r1(g), _r1(b), w)


def _p35_body(a_ref, mu_ref, var_ref, g_ref, b_ref, o_ref):
    o_ref[...] = _bn_apply(a_ref[...], mu_ref[...], var_ref[...],
                           g_ref[...], b_ref[...]).T


def _affine_transpose(a3, stats, g, b):
    sm = lambda shape: pl.BlockSpec(shape, lambda i: (0, 0))
    return pl.pallas_call(
        _p35_body,
        grid=(NB,),
        in_specs=[pl.BlockSpec((EB, 64), lambda i: (i, 0)),
                  sm((1, 64)), sm((1, 64)), sm((1, 64)), sm((1, 64))],
        out_specs=pl.BlockSpec((64, EB), lambda i: (0, i)),
        out_shape=jax.ShapeDtypeStruct((64, E), jnp.float32),
    )(a3, stats[0], stats[1], _r1(g), _r1(b))


def _agg_mean_body(sp_ref, mu_ref, var_ref, g_ref, b_ref, w_ref,
                   z_ref, cnt_ref):
    tot = sp_ref[0] + sp_ref[1]
    cnt = tot[:N, 64:65]
    cnt_ref[...] = cnt
    ssum = tot[:N, :64]
    mean = ssum / jnp.maximum(cnt, 1.0)
    agg = jnp.where(cnt > 0,
                    _bn_apply(mean, mu_ref[...], var_ref[...],
                              g_ref[...], b_ref[...]), 0.0)
    z_ref[...] = _dot(agg, w_ref[...])


def _agg_mean(sumpart, stats, g, b, w):
    return pl.pallas_call(
        _agg_mean_body,
        out_shape=[jax.ShapeDtypeStruct((N, w.shape[1]), jnp.float32),
                   jax.ShapeDtypeStruct((N, 1), jnp.float32)],
    )(sumpart, stats[0], stats[1], _r1(g), _r1(b), w)


def _agg_max_body(mx_ref, cnt_ref, w_ref, z_ref):
    agg = jnp.where(cnt_ref[...] > 0, mx_ref[...].T, 0.0)
    z_ref[...] = _dot(agg, w_ref[...])


def _agg_max(maxT, cnt, w):
    return pl.pallas_call(
        _agg_max_body,
        out_shape=jax.ShapeDtypeStruct((N, w.shape[1]), jnp.float32),
    )(maxT, cnt, w)


def _bn_only_body(z_ref, mu_ref, var_ref, g_ref, b_ref, o_ref, act):
    o = _bn_apply(z_ref[...], mu_ref[...], var_ref[...],
                  g_ref[...], b_ref[...])
    if act == 'tanh':
        o = jnp.tanh(o)
    o_ref[...] = o


def _bn_only(z, stats, g, b, act='none'):
    return pl.pallas_call(
        functools.partial(_bn_only_body, act=act),
        out_shape=jax.ShapeDtypeStruct(z.shape, jnp.float32),
    )(z, stats[0], stats[1], _r1(g), _r1(b))


def _gate1_body(x1_ref, x2_ref, x3_ref, w_ref, o_ref):
    h = jnp.concatenate([x1_ref[...], x2_ref[...], x3_ref[...]], axis=1)
    o_ref[...] = _dot(h, w_ref[...])


def _gate1(x1, x2, x3, w):
    return pl.pallas_call(
        _gate1_body,
        out_shape=jax.ShapeDtypeStruct((N, w.shape[1]), jnp.float32),
    )(x1, x2, x3, w)


def _pool_body(x1_ref, x2_ref, x3_ref, z_ref, mu_ref, var_ref, g_ref, b_ref,
               b2_ref, mw_ref, o_ref):
    h = jnp.concatenate([x1_ref[...], x2_ref[...], x3_ref[...]], axis=1)
    gate = _bn_apply(z_ref[...], mu_ref[...], var_ref[...],
                     g_ref[...], b_ref[...])
    batch = b2_ref[...]
    gid = lax.broadcasted_iota(jnp.int32, (1, NG), 1)
    onehot = (batch == gid).astype(jnp.float32)
    masked = jnp.where(onehot > 0, gate, NEG)
    gmax = jnp.max(masked, axis=0, keepdims=True)
    gmax = jnp.where(gmax > -1e38, gmax, 0.0)
    ge = jnp.exp(gate - _dot_hp(onehot, gmax.T, ((1,), (0,))))
    denom = _dot_hp(onehot, ge, ((0,), (0,)))
    alpha = ge / jnp.maximum(_dot_hp(onehot, denom, ((1,), (0,))), 1e-16)
    pooled = _dot_hp(onehot, alpha * h, ((0,), (0,)))
    o_ref[...] = _dot(pooled, mw_ref[...])


def _pool(x1, x2, x3, zg, stats, g, b, batch, mw):
    return pl.pallas_call(
        _pool_body,
        out_shape=jax.ShapeDtypeStruct((NG, mw.shape[1]), jnp.float32),
    )(x1, x2, x3, zg, stats[0], stats[1], _r1(g), _r1(b),
      batch.reshape(N, 1).astype(jnp.int32), mw)


def _gather_kernel_body(u_hbm, v_hbm, src_hbm, dst_hbm, gu_hbm, gv_hbm,
                        idxs_v, idxd_v, rows_v, rowd_v, sem, sem2):
    wid = lax.axis_index("s") * 2 + lax.axis_index("c")
    base = wid * EW

    def body(i, carry):
        off = pl.multiple_of(base + i * CH, 8)
        pltpu.sync_copy(src_hbm.at[pl.ds(off, CH)], idxs_v)
        pltpu.sync_copy(dst_hbm.at[pl.ds(off, CH)], idxd_v)
        cu = pltpu.async_copy(u_hbm.at[idxs_v], rows_v, sem)
        cv = pltpu.async_copy(v_hbm.at[idxd_v], rowd_v, sem2)
        cu.wait()
        pltpu.sync_copy(rows_v, gu_hbm.at[pl.ds(off, CH)])
        cv.wait()
        pltpu.sync_copy(rowd_v, gv_hbm.at[pl.ds(off, CH)])
        return carry

    lax.fori_loop(0, EW // CH, body, 0)


@functools.cache
def _get_sc_gather(d):
    return functools.partial(
        pl.kernel, mesh=_mesh(),
        out_type=[jax.ShapeDtypeStruct((E, d), jnp.float32),
                  jax.ShapeDtypeStruct((E, d), jnp.float32)],
        scratch_types=[pltpu.VMEM((CH,), jnp.int32),
                       pltpu.VMEM((CH,), jnp.int32),
                       pltpu.VMEM((CH, d), jnp.float32),
                       pltpu.VMEM((CH, d), jnp.float32),
                       pltpu.SemaphoreType.DMA,
                       pltpu.SemaphoreType.DMA],
    )(_gather_kernel_body)


def _scatter_add_body(a3_hbm, dst_hbm, z_hbm, sum_hbm, idx_v, rows_v, acc_sp):
    c = lax.axis_index("c")
    s = lax.axis_index("s")
    wid = s * 2 + c
    base = wid * EW
    pltpu.sync_copy(z_hbm, acc_sp.at[pl.ds(s * RPT, RPT)])
    plsc.subcore_barrier()

    def loop(i, carry):
        off = pl.multiple_of(base + i * CH, 8)
        pltpu.sync_copy(dst_hbm.at[pl.ds(off, CH)], idx_v)
        pltpu.sync_copy(a3_hbm.at[pl.ds(off, CH)], rows_v)
        pltpu.sync_copy(rows_v, acc_sp.at[idx_v], add=True)
        return carry

    lax.fori_loop(0, EW // CH, loop, 0)
    plsc.subcore_barrier()
    pltpu.sync_copy(acc_sp.at[pl.ds(s * RPT, RPT)],
                    sum_hbm.at[c, pl.ds(s * RPT, RPT)])


@functools.cache
def _get_sc_scatter_add():
    return functools.partial(
        pl.kernel, mesh=_mesh(),
        out_type=jax.ShapeDtypeStruct((2, NP, 128), jnp.float32),
        scratch_types=[pltpu.VMEM((CH,), jnp.int32),
                       pltpu.VMEM((CH, 128), jnp.float32),
                       pltpu.VMEM_SHARED((NP, 128), jnp.float32)],
    )(_scatter_add_body)


def _rmw_max(acc, keys, vals):
    cur = plsc.load_gather(acc, [keys])
    plsc.store_scatter(acc, [keys], jnp.maximum(cur, vals))
    chk = plsc.load_gather(acc, [keys])
    pend = chk < vals

    def cond(p):
        return jnp.any(p)

    def body(p):
        c2 = plsc.load_gather(acc, [keys])
        plsc.store_scatter(acc, [keys], jnp.maximum(c2, vals), mask=p)
        c3 = plsc.load_gather(acc, [keys])
        return c3 < vals

    lax.while_loop(cond, body, pend)


_MC = 2000


def _scatter_max_body(m_hbm, dst_hbm, neg_hbm, out_hbm,
                      idx_v, v0_v, v1_v, acc0, acc1):
    wid = lax.axis_index("s") * 2 + lax.axis_index("c")
    f0 = wid * 2
    pltpu.sync_copy(neg_hbm, acc0)
    pltpu.sync_copy(neg_hbm, acc1)

    def outer(i, carry):
        off = pl.multiple_of(i * _MC, 8)
        pltpu.sync_copy(dst_hbm.at[pl.ds(off, _MC)], idx_v)
        pltpu.sync_copy(m_hbm.at[pl.ds(pl.multiple_of(f0 * E, 8) + off, _MC)], v0_v)
        pltpu.sync_copy(m_hbm.at[pl.ds(pl.multiple_of((f0 + 1) * E, 8) + off, _MC)], v1_v)

        def inner(j, c2):
            keys = idx_v[pl.ds(j * 16, 16)]
            _rmw_max(acc0, keys, v0_v[pl.ds(j * 16, 16)])
            _rmw_max(acc1, keys, v1_v[pl.ds(j * 16, 16)])
            return c2

        lax.fori_loop(0, _MC // 16, inner, 0)
        return carry

    lax.fori_loop(0, E // _MC, outer, 0)
    pltpu.sync_copy(acc0, out_hbm.at[pl.ds(pl.multiple_of(f0 * N, 8), N)])
    pltpu.sync_copy(acc1, out_hbm.at[pl.ds(pl.multiple_of((f0 + 1) * N, 8), N)])


@functools.cache
def _get_sc_scatter_max():
    return functools.partial(
        pl.kernel, mesh=_mesh(),
        compiler_params=pltpu.CompilerParams(needs_layout_passes=False),
        out_type=jax.ShapeDtypeStruct((64 * N,), jnp.float32),
        scratch_types=[pltpu.VMEM((_MC,), jnp.int32),
                       pltpu.VMEM((_MC,), jnp.float32),
                       pltpu.VMEM((_MC,), jnp.float32),
                       pltpu.VMEM((N,), jnp.float32),
                       pltpu.VMEM((N,), jnp.float32)],
    )(_scatter_max_body)


def _xb_chain(x_in, p):
    nx = p['net_x']
    z1 = _bnmm(x_in, _mv(x_in), p['bn_x']['gamma'], p['bn_x']['beta'],
               nx[0]['W'])
    z2 = _bnmm(z1, _mv(z1), nx[0]['gamma'], nx[0]['beta'], nx[1]['W'],
               act='lrelu')
    return _bn_only(z2, _mv(z2), nx[1]['gamma'], nx[1]['beta'], act='tanh')


def _fmlp_chain(z1, f):
    z2 = _bnmm(z1, _mv(z1), f[0]['gamma'], f[0]['beta'], f[1]['W'],
               act='lrelu')
    z3 = _bnmm(z2, _mv(z2), f[1]['gamma'], f[1]['beta'], f[2]['W'],
               act='lrelu')
    return _bn_only(z3, _mv(z3), f[2]['gamma'], f[2]['beta'])


def kernel(x, edge_attr, params, edge_index, batch):
    src = edge_index[0].astype(jnp.int32)
    dst = edge_index[1].astype(jnp.int32)
    convs = [params['conv1'], params['conv2'], params['conv3']]

    e_stats = _mv(edge_attr)
    z1s = _edge_pass(edge_attr,
                     [(e_stats[0], e_stats[1], p['bn_e']['gamma'],
                       p['bn_e']['beta'], p['net_e'][0]['W']) for p in convs],
                     act='none', dout=32)
    z1_stats = [_mv(z) for z in z1s]
    z2s = []
    for c in range(3):
        p = convs[c]
        z2s.append(_mid_pass(z1s[c], z1_stats[c], p['net_e'][0]['gamma'],
                             p['net_e'][0]['beta'], p['net_e'][1]['W']))
    z2_stats = [_mv(z) for z in z2s]

    zrows = jnp.zeros((RPT, 128), jnp.float32)
    negfill = jnp.full((N,), NEG, jnp.float32)

    xs = []
    cnt = None
    cntpart = None
    x_in = x
    for ci in range(3):
        p = convs[ci]
        xb = _xb_chain(x_in, p)
        d = xb.shape[1]
        xbg = xb if d == 128 else jnp.pad(xb, ((0, 0), (0, 128 - d)))
        gu, gv = _get_sc_gather(128)(xbg, xbg, src, dst)
        a1 = _pass1(z2s[ci], gu, gv, z2_stats[ci], p['net_e'][1]['gamma'],
                    p['net_e'][1]['beta'], p['g'][0]['W'], d)
        a2 = _mid_pass(a1, _mv(a1), p['g'][0]['gamma'], p['g'][0]['beta'],
                       p['g'][1]['W'])
        g3, b3 = p['g'][2]['gamma'], p['g'][2]['beta']
        if ci == 1:
            a3 = _mid_pass(a2, _mv(a2), p['g'][1]['gamma'], p['g'][1]['beta'],
                           p['g'][2]['W'])
            st3 = _mv(a3)
            m3T = _affine_transpose(a3, st3, g3, b3)
            maxT = _get_sc_scatter_max()(m3T.reshape(-1), dst, negfill)
            zf1 = _agg_max(maxT.reshape(64, N), cnt, p['f'][0]['W'])
        else:
            a3p = _mid_pass(a2, _mv(a2), p['g'][1]['gamma'], p['g'][1]['beta'],
                            p['g'][2]['W'], pad_ones=True)
            st3 = _mv(a3p[:, :64])
            res = _get_sc_scatter_add()(a3p, dst, zrows)
            sumpart = res[0] if isinstance(res, (list, tuple)) else res
            zf1, cnt_new = _agg_mean(sumpart, st3, g3, b3, p['f'][0]['W'])
            if cnt is None:
                cnt = cnt_new
        xs.append(_fmlp_chain(zf1, p['f']))
        if ci == 0:
            x_in = xs[0]
        elif ci == 1:
            x_in = jnp.concatenate([xs[0], xs[1]], axis=1)

    x1, x2, x3 = xs
    gn = params['gate_nn']
    zg1 = _gate1(x1, x2, x3, gn[0]['W'])
    zg2 = _bnmm(zg1, _mv(zg1), gn[0]['gamma'], gn[0]['beta'], gn[1]['W'],
                act='lrelu')
    zg3 = _bnmm(zg2, _mv(zg2), gn[1]['gamma'], gn[1]['beta'], gn[2]['W'],
                act='lrelu')
    mlp = params['mlp']
    zm = _pool(x1, x2, x3, zg3, _mv(zg3), gn[2]['gamma'], gn[2]['beta'],
               batch, mlp[0]['W'])
    return _bn_only(zm, _mv(zm), mlp[0]['gamma'], mlp[0]['beta'])


# speedup vs baseline: 1.0973x; 1.0002x over previous
import functools

import jax
import jax.numpy as jnp
from jax import lax
from jax.experimental import pallas as pl
from jax.experimental.pallas import tpu as pltpu
from jax.experimental.pallas import tpu_sc as plsc

N = 10000
E = 320000
NG = 16
EPS = 1e-5
EB = 2560
NB = E // EB
CH = 80
EW = E // 32
NEG = -3.0e38
NP = 10240
RPT = NP // 16


@functools.cache
def _mesh():
    return plsc.VectorSubcoreMesh(core_axis_name="c", subcore_axis_name="s")


def _dot(a, b):
    return jnp.dot(a, b, preferred_element_type=jnp.float32)


def _dot_hp(a, b, dims):
    return lax.dot_general(a, b, (dims, ((), ())),
                           precision=lax.Precision.HIGHEST,
                           preferred_element_type=jnp.float32)


def _lrelu(x):
    return jnp.where(x > 0, x, 0.2 * x)


def _bn_apply(z, mu, var, gamma, beta):
    return (z - mu) / jnp.sqrt(var + EPS) * gamma + beta


def _mv(z):
    return (jnp.mean(z, axis=0, keepdims=True),
            jnp.var(z, axis=0, keepdims=True))


def _r1(a):
    return a.reshape(1, -1)


def _bnmm_body(x_ref, mu_ref, var_ref, g_ref, b_ref, w_ref, o_ref, act):
    z = _bn_apply(x_ref[...], mu_ref[...], var_ref[...], g_ref[...], b_ref[...])
    if act == 'lrelu':
        z = _lrelu(z)
    elif act == 'tanh':
        z = jnp.tanh(z)
    o_ref[...] = _dot(z, w_ref[...])


def _bnmm(x, stats, g, b, w, act='none'):
    mu, var = stats
    body = functools.partial(_bnmm_body, act=act)
    return pl.pallas_call(
        body,
        out_shape=jax.ShapeDtypeStruct((x.shape[0], w.shape[1]), jnp.float32),
    )(x, mu, var, _r1(g), _r1(b), w)


def _ep_body(x_ref, p_refs, o_refs, act, nconv):
    x = x_ref[...]
    for c in range(nconv):
        mu, var, g, b, w = p_refs[5 * c:5 * c + 5]
        z = _bn_apply(x, mu[...], var[...], g[...], b[...])
        if act == 'lrelu':
            z = _lrelu(z)
        elif act == 'tanh':
            z = jnp.tanh(z)
        o_refs[c][...] = _dot(z, w[...])


def _edge_pass(x, params3, act, dout):
    din = x.shape[1]
    nconv = len(params3)

    def body(*refs):
        x_ref = refs[0]
        p_refs = refs[1:1 + 5 * nconv]
        o_refs = refs[1 + 5 * nconv:]
        _ep_body(x_ref, p_refs, o_refs, act, nconv)

    sm = lambda shape: pl.BlockSpec(shape, lambda i: (0, 0))
    in_specs = [pl.BlockSpec((EB, din), lambda i: (i, 0))]
    args = [x]
    for (mu, var, g, b, w) in params3:
        in_specs += [sm((1, din)), sm((1, din)), sm((1, din)), sm((1, din)),
                     sm(w.shape)]
        args += [mu, var, _r1(g), _r1(b), w]
    return pl.pallas_call(
        body,
        grid=(NB,),
        in_specs=in_specs,
        out_specs=[pl.BlockSpec((EB, dout), lambda i: (i, 0))] * nconv,
        out_shape=[jax.ShapeDtypeStruct((E, dout), jnp.float32)] * nconv,
    )(*args)


def _pass1_body(z2_ref, gu_ref, gv_ref, mu_ref, var_ref, g_ref, b_ref,
                w1_ref, a1_ref, d):
    eb = jnp.tanh(_bn_apply(z2_ref[...], mu_ref[...], var_ref[...],
                            g_ref[...], b_ref[...]))
    cat = jnp.concatenate([eb, gu_ref[...][:, :d], gv_ref[...][:, :d]], axis=1)
    a1_ref[...] = _dot(cat, w1_ref[...])


def _pass1(z2, gu, gv, stats, g, b, w1, d):
    mu, var = stats
    sm = lambda shape: pl.BlockSpec(shape, lambda i: (0, 0))
    return pl.pallas_call(
        functools.partial(_pass1_body, d=d),
        grid=(NB,),
        in_specs=[pl.BlockSpec((EB, 16), lambda i: (i, 0)),
                  pl.BlockSpec((EB, 128), lambda i: (i, 0)),
                  pl.BlockSpec((EB, 128), lambda i: (i, 0)),
                  sm((1, 16)), sm((1, 16)), sm((1, 16)), sm((1, 16)),
                  sm(w1.shape)],
        out_specs=pl.BlockSpec((EB, 128), lambda i: (i, 0)),
        out_shape=jax.ShapeDtypeStruct((E, 128), jnp.float32),
    )(z2, gu, gv, mu, var, _r1(g), _r1(b), w1)


def _mid_body(a_ref, mu_ref, var_ref, g_ref, b_ref, w_ref, o_ref, pad_ones):
    h = _lrelu(_bn_apply(a_ref[...], mu_ref[...], var_ref[...],
                         g_ref[...], b_ref[...]))
    o = _dot(h, w_ref[...])
    if pad_ones:
        o = jnp.concatenate(
            [o, jnp.ones((o.shape[0], 1), jnp.float32),
             jnp.zeros((o.shape[0], 127 - o.shape[1]), jnp.float32)], axis=1)
    o_ref[...] = o


def _mid_pass(a, stats, g, b, w, pad_ones=False):
    mu, var = stats
    din, dout = w.shape
    ow = 128 if pad_ones else dout
    sm = lambda shape: pl.BlockSpec(shape, lambda i: (0, 0))
    return pl.pallas_call(
        functools.partial(_mid_body, pad_ones=pad_ones),
        grid=(NB,),
        in_specs=[pl.BlockSpec((EB, din), lambda i: (i, 0)),
                  sm((1, din)), sm((1, din)), sm((1, din)), sm((1, din)),
                  sm((din, dout))],
        out_specs=pl.BlockSpec((EB, ow), lambda i: (i, 0)),
        out_shape=jax.ShapeDtypeStruct((E, ow), jnp.float32),
    )(a, stats[0], stats[1], _r1(g), _r1(b), w)


def _p35_body(a_ref, mu_ref, var_ref, g_ref, b_ref, o_ref):
    o_ref[...] = _bn_apply(a_ref[...], mu_ref[...], var_ref[...],
                           g_ref[...], b_ref[...]).T


def _affine_transpose(a3, stats, g, b):
    sm = lambda shape: pl.BlockSpec(shape, lambda i: (0, 0))
    return pl.pallas_call(
        _p35_body,
        grid=(NB,),
        in_specs=[pl.BlockSpec((EB, 64), lambda i: (i, 0)),
                  sm((1, 64)), sm((1, 64)), sm((1, 64)), sm((1, 64))],
        out_specs=pl.BlockSpec((64, EB), lambda i: (0, i)),
        out_shape=jax.ShapeDtypeStruct((64, E), jnp.float32),
    )(a3, stats[0], stats[1], _r1(g), _r1(b))


def _agg_mean_body(sp_ref, mu_ref, var_ref, g_ref, b_ref, w_ref,
                   z_ref, cnt_ref):
    tot = sp_ref[0] + sp_ref[1]
    cnt = tot[:N, 64:65]
    cnt_ref[...] = cnt
    ssum = tot[:N, :64]
    mean = ssum / jnp.maximum(cnt, 1.0)
    agg = jnp.where(cnt > 0,
                    _bn_apply(mean, mu_ref[...], var_ref[...],
                              g_ref[...], b_ref[...]), 0.0)
    z_ref[...] = _dot(agg, w_ref[...])


def _agg_mean(sumpart, stats, g, b, w):
    return pl.pallas_call(
        _agg_mean_body,
        out_shape=[jax.ShapeDtypeStruct((N, w.shape[1]), jnp.float32),
                   jax.ShapeDtypeStruct((N, 1), jnp.float32)],
    )(sumpart, stats[0], stats[1], _r1(g), _r1(b), w)


def _agg_max_body(mx_ref, cnt_ref, w_ref, z_ref):
    agg = jnp.where(cnt_ref[...] > 0, mx_ref[...].T, 0.0)
    z_ref[...] = _dot(agg, w_ref[...])


def _agg_max(maxT, cnt, w):
    return pl.pallas_call(
        _agg_max_body,
        out_shape=jax.ShapeDtypeStruct((N, w.shape[1]), jnp.float32),
    )(maxT, cnt, w)


def _bn_only_body(z_ref, mu_ref, var_ref, g_ref, b_ref, o_ref, act):
    o = _bn_apply(z_ref[...], mu_ref[...], var_ref[...],
                  g_ref[...], b_ref[...])
    if act == 'tanh':
        o = jnp.tanh(o)
    o_ref[...] = o


def _bn_only(z, stats, g, b, act='none'):
    return pl.pallas_call(
        functools.partial(_bn_only_body, act=act),
        out_shape=jax.ShapeDtypeStruct(z.shape, jnp.float32),
    )(z, stats[0], stats[1], _r1(g), _r1(b))


def _gate1_body(x1_ref, x2_ref, x3_ref, w_ref, o_ref):
    h = jnp.concatenate([x1_ref[...], x2_ref[...], x3_ref[...]], axis=1)
    o_ref[...] = _dot(h, w_ref[...])


def _gate1(x1, x2, x3, w):
    return pl.pallas_call(
        _gate1_body,
        out_shape=jax.ShapeDtypeStruct((N, w.shape[1]), jnp.float32),
    )(x1, x2, x3, w)


def _pool_body(x1_ref, x2_ref, x3_ref, z_ref, mu_ref, var_ref, g_ref, b_ref,
               b2_ref, mw_ref, o_ref):
    h = jnp.concatenate([x1_ref[...], x2_ref[...], x3_ref[...]], axis=1)
    gate = _bn_apply(z_ref[...], mu_ref[...], var_ref[...],
                     g_ref[...], b_ref[...])
    batch = b2_ref[...]
    gid = lax.broadcasted_iota(jnp.int32, (1, NG), 1)
    onehot = (batch == gid).astype(jnp.float32)
    masked = jnp.where(onehot > 0, gate, NEG)
    gmax = jnp.max(masked, axis=0, keepdims=True)
    gmax = jnp.where(gmax > -1e38, gmax, 0.0)
    ge = jnp.exp(gate - _dot_hp(onehot, gmax.T, ((1,), (0,))))
    denom = _dot_hp(onehot, ge, ((0,), (0,)))
    alpha = ge / jnp.maximum(_dot_hp(onehot, denom, ((1,), (0,))), 1e-16)
    pooled = _dot_hp(onehot, alpha * h, ((0,), (0,)))
    o_ref[...] = _dot(pooled, mw_ref[...])


def _pool(x1, x2, x3, zg, stats, g, b, batch, mw):
    return pl.pallas_call(
        _pool_body,
        out_shape=jax.ShapeDtypeStruct((NG, mw.shape[1]), jnp.float32),
    )(x1, x2, x3, zg, stats[0], stats[1], _r1(g), _r1(b),
      batch.reshape(N, 1).astype(jnp.int32), mw)


def _gather_kernel_body(u_hbm, v_hbm, src_hbm, dst_hbm, gu_hbm, gv_hbm,
                        idxs_v, idxd_v, rows_v, rowd_v, sem, sem2):
    wid = lax.axis_index("s") * 2 + lax.axis_index("c")
    base = wid * EW

    def body(i, carry):
        off = pl.multiple_of(base + i * CH, 8)
        pltpu.sync_copy(src_hbm.at[pl.ds(off, CH)], idxs_v)
        pltpu.sync_copy(dst_hbm.at[pl.ds(off, CH)], idxd_v)
        cu = pltpu.async_copy(u_hbm.at[idxs_v], rows_v, sem)
        cv = pltpu.async_copy(v_hbm.at[idxd_v], rowd_v, sem2)
        cu.wait()
        pltpu.sync_copy(rows_v, gu_hbm.at[pl.ds(off, CH)])
        cv.wait()
        pltpu.sync_copy(rowd_v, gv_hbm.at[pl.ds(off, CH)])
        return carry

    lax.fori_loop(0, EW // CH, body, 0)


@functools.cache
def _get_sc_gather(d):
    return functools.partial(
        pl.kernel, mesh=_mesh(),
        out_type=[jax.ShapeDtypeStruct((E, d), jnp.float32),
                  jax.ShapeDtypeStruct((E, d), jnp.float32)],
        scratch_types=[pltpu.VMEM((CH,), jnp.int32),
                       pltpu.VMEM((CH,), jnp.int32),
                       pltpu.VMEM((CH, d), jnp.float32),
                       pltpu.VMEM((CH, d), jnp.float32),
                       pltpu.SemaphoreType.DMA,
                       pltpu.SemaphoreType.DMA],
    )(_gather_kernel_body)


def _scatter_add_body(a3_hbm, dst_hbm, z_hbm, sum_hbm, idx_v, rows_v, acc_sp):
    c = lax.axis_index("c")
    s = lax.axis_index("s")
    wid = s * 2 + c
    base = wid * EW
    pltpu.sync_copy(z_hbm, acc_sp.at[pl.ds(s * RPT, RPT)])
    plsc.subcore_barrier()

    def loop(i, carry):
        off = pl.multiple_of(base + i * CH, 8)
        pltpu.sync_copy(dst_hbm.at[pl.ds(off, CH)], idx_v)
        pltpu.sync_copy(a3_hbm.at[pl.ds(off, CH)], rows_v)
        pltpu.sync_copy(rows_v, acc_sp.at[idx_v], add=True)
        return carry

    lax.fori_loop(0, EW // CH, loop, 0)
    plsc.subcore_barrier()
    pltpu.sync_copy(acc_sp.at[pl.ds(s * RPT, RPT)],
                    sum_hbm.at[c, pl.ds(s * RPT, RPT)])


@functools.cache
def _get_sc_scatter_add():
    return functools.partial(
        pl.kernel, mesh=_mesh(),
        out_type=jax.ShapeDtypeStruct((2, NP, 128), jnp.float32),
        scratch_types=[pltpu.VMEM((CH,), jnp.int32),
                       pltpu.VMEM((CH, 128), jnp.float32),
                       pltpu.VMEM_SHARED((NP, 128), jnp.float32)],
    )(_scatter_add_body)


def _rmw_max(acc, keys, vals):
    cur = plsc.load_gather(acc, [keys])
    plsc.store_scatter(acc, [keys], jnp.maximum(cur, vals))
    chk = plsc.load_gather(acc, [keys])
    pend = chk < vals

    def cond(p):
        return jnp.any(p)

    def body(p):
        c2 = plsc.load_gather(acc, [keys])
        plsc.store_scatter(acc, [keys], jnp.maximum(c2, vals), mask=p)
        c3 = plsc.load_gather(acc, [keys])
        return c3 < vals

    lax.while_loop(cond, body, pend)


_MC = 2000


def _scatter_max_body(m_hbm, dst_hbm, neg_hbm, out_hbm,
                      idx_v, v0_v, v1_v, acc0, acc1):
    wid = lax.axis_index("s") * 2 + lax.axis_index("c")
    f0 = wid * 2
    pltpu.sync_copy(neg_hbm, acc0)
    pltpu.sync_copy(neg_hbm, acc1)

    def outer(i, carry):
        off = pl.multiple_of(i * _MC, 8)
        pltpu.sync_copy(dst_hbm.at[pl.ds(off, _MC)], idx_v)
        pltpu.sync_copy(m_hbm.at[pl.ds(pl.multiple_of(f0 * E, 8) + off, _MC)], v0_v)
        pltpu.sync_copy(m_hbm.at[pl.ds(pl.multiple_of((f0 + 1) * E, 8) + off, _MC)], v1_v)

        def inner(j, c2):
            keys = idx_v[pl.ds(j * 16, 16)]
            _rmw_max(acc0, keys, v0_v[pl.ds(j * 16, 16)])
            _rmw_max(acc1, keys, v1_v[pl.ds(j * 16, 16)])
            return c2

        lax.fori_loop(0, _MC // 16, inner, 0)
        return carry

    lax.fori_loop(0, E // _MC, outer, 0)
    pltpu.sync_copy(acc0, out_hbm.at[pl.ds(pl.multiple_of(f0 * N, 8), N)])
    pltpu.sync_copy(acc1, out_hbm.at[pl.ds(pl.multiple_of((f0 + 1) * N, 8), N)])


@functools.cache
def _get_sc_scatter_max():
    return functools.partial(
        pl.kernel, mesh=_mesh(),
        compiler_params=pltpu.CompilerParams(needs_layout_passes=False),
        out_type=jax.ShapeDtypeStruct((64 * N,), jnp.float32),
        scratch_types=[pltpu.VMEM((_MC,), jnp.int32),
                       pltpu.VMEM((_MC,), jnp.float32),
                       pltpu.VMEM((_MC,), jnp.float32),
                       pltpu.VMEM((N,), jnp.float32),
                       pltpu.VMEM((N,), jnp.float32)],
    )(_scatter_max_body)


def _xb_chain(x_in, p):
    nx = p['net_x']
    z1 = _bnmm(x_in, _mv(x_in), p['bn_x']['gamma'], p['bn_x']['beta'],
               nx[0]['W'])
    z2 = _bnmm(z1, _mv(z1), nx[0]['gamma'], nx[0]['beta'], nx[1]['W'],
               act='lrelu')
    return _bn_only(z2, _mv(z2), nx[1]['gamma'], nx[1]['beta'], act='tanh')


def _fmlp_chain(z1, f):
    z2 = _bnmm(z1, _mv(z1), f[0]['gamma'], f[0]['beta'], f[1]['W'],
               act='lrelu')
    z3 = _bnmm(z2, _mv(z2), f[1]['gamma'], f[1]['beta'], f[2]['W'],
               act='lrelu')
    return _bn_only(z3, _mv(z3), f[2]['gamma'], f[2]['beta'])


def kernel(x, edge_attr, params, edge_index, batch):
    src = edge_index[0].astype(jnp.int32)
    dst = edge_index[1].astype(jnp.int32)
    convs = [params['conv1'], params['conv2'], params['conv3']]

    e_stats = _mv(edge_attr)
    z1s = _edge_pass(edge_attr,
                     [(e_stats[0], e_stats[1], p['bn_e']['gamma'],
                       p['bn_e']['beta'], p['net_e'][0]['W']) for p in convs],
                     act='none', dout=32)
    z1_stats = [_mv(z) for z in z1s]
    z2s = []
    for c in range(3):
        p = convs[c]
        z2s.append(_mid_pass(z1s[c], z1_stats[c], p['net_e'][0]['gamma'],
                             p['net_e'][0]['beta'], p['net_e'][1]['W']))
    z2_stats = [_mv(z) for z in z2s]

    zrows = jnp.zeros((RPT, 128), jnp.float32)
    negfill = jnp.full((N,), NEG, jnp.float32)

    xs = []
    cnt = None
    cntpart = None
    x_in = x
    for ci in range(3):
        p = convs[ci]
        xb = _xb_chain(x_in, p)
        d = xb.shape[1]
        xbg = xb if d == 128 else jnp.pad(xb, ((0, 0), (0, 128 - d)))
        gu, gv = _get_sc_gather(128)(xbg, xbg, src, dst)
        a1 = _pass1(z2s[ci], gu, gv, z2_stats[ci], p['net_e'][1]['gamma'],
                    p['net_e'][1]['beta'], p['g'][0]['W'], d)
        a2 = _mid_pass(a1, _mv(a1), p['g'][0]['gamma'], p['g'][0]['beta'],
                       p['g'][1]['W'])
        g3, b3 = p['g'][2]['gamma'], p['g'][2]['beta']
        if ci == 1:
            a3 = _mid_pass(a2, _mv(a2), p['g'][1]['gamma'], p['g'][1]['beta'],
                           p['g'][2]['W'])
            st3 = _mv(a3)
            m3T = _affine_transpose(a3, st3, g3, b3)
            maxT = _get_sc_scatter_max()(m3T.reshape(-1), dst, negfill)
            zf1 = _agg_max(maxT.reshape(64, N), cnt, p['f'][0]['W'])
        else:
            a3p = _mid_pass(a2, _mv(a2), p['g'][1]['gamma'], p['g'][1]['beta'],
                            p['g'][2]['W'], pad_ones=True)
            st3 = _mv(a3p[:, :64])
            res = _get_sc_scatter_add()(a3p, dst, zrows)
            sumpart = res[0] if isinstance(res, (list, tuple)) else res
            zf1, cnt_new = _agg_mean(sumpart, st3, g3, b3, p['f'][0]['W'])
            if cnt is None:
                cnt = cnt_new
        xs.append(_fmlp_chain(zf1, p['f']))
        if ci == 0:
            x_in = xs[0]
        elif ci == 1:
            x_in = jnp.concatenate([xs[0], xs[1]], axis=1)

    x1, x2, x3 = xs
    gn = params['gate_nn']
    zg1 = _gate1(x1, x2, x3, gn[0]['W'])
    zg2 = _bnmm(zg1, _mv(zg1), gn[0]['gamma'], gn[0]['beta'], gn[1]['W'],
                act='lrelu')
    zg3 = _bnmm(zg2, _mv(zg2), gn[1]['gamma'], gn[1]['beta'], gn[2]['W'],
                act='lrelu')
    mlp = params['mlp']
    zm = _pool(x1, x2, x3, zg3, _mv(zg3), gn[2]['gamma'], gn[2]['beta'],
               batch, mlp[0]['W'])
    return _bn_only(zm, _mv(zm), mlp[0]['gamma'], mlp[0]['beta'])
